# Initial kernel scaffold; baseline (speedup 1.0000x reference)
import jax
import jax.numpy as jnp
from jax.experimental import pallas as pl


def kernel(x, edge_index, W1, b1, gamma, beta, W2, b2):
    raise NotImplementedError("write your pallas kernel here")


# speedup vs baseline: 12.1729x; 12.1729x over previous
import functools

import jax
import jax.numpy as jnp
from jax import lax
from jax.experimental import pallas as pl
from jax.experimental.pallas import tpu as pltpu
from jax.experimental.pallas import tpu_sc as plsc

N = 10000
D = 128
BN_EPS = 1e-5

NUM_CORES = 2
NUM_SUBCORES = 16
NUM_TILES = NUM_CORES * NUM_SUBCORES

W = 128
KW = 8
E_IN = 320000
STEP = KW * W
CPT = -(-E_IN // (STEP * NUM_TILES))
EPAD = NUM_TILES * CPT * STEP
NPAD = 10112
RPT = NPAD // NUM_SUBCORES

@jax.jit
def _sc_aggregate(x, src, dst):
    _vector_mesh = plsc.VectorSubcoreMesh(
        core_axis_name="core", subcore_axis_name="subcore",
        num_cores=NUM_CORES, num_subcores=NUM_SUBCORES)
    src4 = src.reshape(NUM_TILES, CPT, KW, W)
    dst4 = dst.reshape(NUM_TILES, CPT, KW, W)

    @functools.partial(
        pl.kernel,
        out_type=jax.ShapeDtypeStruct((NUM_CORES, NPAD, D), jnp.float32),
        mesh=_vector_mesh,
        scratch_types=[
            pltpu.VMEM_SHARED((NPAD, D), jnp.float32),
            pltpu.VMEM((W, D), jnp.float32),
            pltpu.VMEM((W, D), jnp.float32),
            pltpu.SemaphoreType.DMA,
            pltpu.SemaphoreType.DMA,
            pltpu.SemaphoreType.DMA,
            pltpu.SemaphoreType.DMA,
        ],
    )
    def agg_kernel(x_hbm, src_hbm, dst_hbm, out_hbm, acc_spmem, rows_vmem,
                   rows_vmem2, gsem_a, gsem_b, ssem_a, ssem_b):
        cid = lax.axis_index("core")
        sid = lax.axis_index("subcore")

        @pl.loop(0, W)
        def _(i):
            for g in range(D // 16):
                rows_vmem[pl.ds(i, 1), pl.ds(g * 16, 16)] = jnp.zeros(
                    (1, 16), jnp.float32)

        zbase = sid * RPT
        for k in range(RPT // W):
            pltpu.sync_copy(rows_vmem, acc_spmem.at[pl.ds(zbase + k * W, W)])
        zrem = RPT % W
        if zrem:
            pltpu.sync_copy(
                rows_vmem.at[pl.ds(0, zrem)],
                acc_spmem.at[pl.ds(zbase + (RPT // W) * W, zrem)])
        plsc.subcore_barrier()

        rows = (rows_vmem, rows_vmem2)
        gsem = (gsem_a, gsem_b)
        ssem = (ssem_a, ssem_b)

        def window(src_idx, dst_idx):
            gd = [None] * KW
            sd = [None] * KW
            gd[0] = pltpu.async_copy(
                x_hbm.at[src_idx.at[0, 0, 0]], rows[0], gsem[0])
            for j in range(KW):
                b = j % 2
                if j + 1 < KW:
                    if j >= 1:
                        sd[j - 1].wait()
                    gd[j + 1] = pltpu.async_copy(
                        x_hbm.at[src_idx.at[0, 0, j + 1]],
                        rows[(j + 1) % 2], gsem[(j + 1) % 2])
                gd[j].wait()
                sd[j] = pltpu.async_copy(
                    rows[b], acc_spmem.at[dst_idx.at[0, 0, j]], ssem[b],
                    add=True)
            sd[KW - 2].wait()
            sd[KW - 1].wait()

        pltpu.emit_pipeline(
            window,
            grid=(NUM_TILES, CPT),
            in_specs=[
                pl.BlockSpec((1, 1, KW, W), lambda t, i: (t, i, 0, 0)),
                pl.BlockSpec((1, 1, KW, W), lambda t, i: (t, i, 0, 0)),
            ],
            core_axis_name=("core", "subcore"),
            dimension_semantics=(pltpu.PARALLEL, pltpu.ARBITRARY),
        )(src_hbm, dst_hbm)

        plsc.subcore_barrier()
        obase = sid * RPT
        pltpu.sync_copy(acc_spmem.at[pl.ds(obase, RPT)],
                        out_hbm.at[cid, pl.ds(obase, RPT)])

    return agg_kernel(x, src4, dst4)


def _mlp_body(x_ref, p_ref, w1t_ref, b1_ref, gamma_ref, beta_ref, w2t_ref,
              b2_ref, o_ref):
    h = x_ref[...] + p_ref[0, pl.ds(0, N)] + p_ref[1, pl.ds(0, N)]
    h1 = jnp.dot(h, w1t_ref[...], preferred_element_type=jnp.float32)
    h1 = h1 + b1_ref[...]
    mean = jnp.mean(h1, axis=0, keepdims=True)
    c = h1 - mean
    var = jnp.mean(c * c, axis=0, keepdims=True)
    hn = c * lax.rsqrt(var + BN_EPS) * gamma_ref[...] + beta_ref[...]
    h2 = jnp.maximum(hn, 0.0)
    o = jnp.dot(h2, w2t_ref[...], preferred_element_type=jnp.float32)
    o = o + b2_ref[...]
    o_ref[...] = jnp.maximum(o, 0.0)


@jax.jit
def _tc_mlp(x, p, w1t, b1, gamma, beta, w2t, b2):
    return pl.pallas_call(
        _mlp_body,
        out_shape=jax.ShapeDtypeStruct((N, D), jnp.float32),
        compiler_params=pltpu.CompilerParams(
            vmem_limit_bytes=100 * 1024 * 1024),
    )(x, p, w1t, b1.reshape(1, D), gamma.reshape(1, D), beta.reshape(1, D),
      w2t, b2.reshape(1, D))


def kernel(x, edge_index, W1, b1, gamma, beta, W2, b2):
    src = edge_index[0].astype(jnp.int32)
    dst = edge_index[1].astype(jnp.int32)
    npad = EPAD - E_IN
    pad_ids = jnp.arange(npad, dtype=jnp.int32)
    src_pad = (pad_ids * 37) % N
    dst_pad = N + pad_ids % (NPAD - N)
    src_full = jnp.concatenate([src, src_pad])
    dst_full = jnp.concatenate([dst, dst_pad])
    p = _sc_aggregate(x, src_full, dst_full)
    return _tc_mlp(x, p, W1.T, b1, gamma, beta, W2.T, b2)


# speedup vs baseline: 12.6006x; 1.0351x over previous
import functools

import jax
import jax.numpy as jnp
from jax import lax
from jax.experimental import pallas as pl
from jax.experimental.pallas import tpu as pltpu
from jax.experimental.pallas import tpu_sc as plsc

N = 10000
D = 128
BN_EPS = 1e-5

NUM_CORES = 2
NUM_SUBCORES = 16
NUM_TILES = NUM_CORES * NUM_SUBCORES

W = 128
KW = 16
E_IN = 320000
STEP = KW * W
CPT = -(-E_IN // (STEP * NUM_TILES))
EPAD = NUM_TILES * CPT * STEP
NPAD = 10112
RPT = NPAD // NUM_SUBCORES

@jax.jit
def _sc_aggregate(x, src, dst):
    _vector_mesh = plsc.VectorSubcoreMesh(
        core_axis_name="core", subcore_axis_name="subcore",
        num_cores=NUM_CORES, num_subcores=NUM_SUBCORES)
    src4 = src.reshape(NUM_TILES, CPT, KW, W)
    dst4 = dst.reshape(NUM_TILES, CPT, KW, W)

    @functools.partial(
        pl.kernel,
        out_type=jax.ShapeDtypeStruct((NUM_CORES, NPAD, D), jnp.float32),
        mesh=_vector_mesh,
        scratch_types=[
            pltpu.VMEM_SHARED((NPAD, D), jnp.float32),
            pltpu.VMEM((W, D), jnp.float32),
            pltpu.VMEM((W, D), jnp.float32),
            pltpu.SemaphoreType.DMA,
            pltpu.SemaphoreType.DMA,
            pltpu.SemaphoreType.DMA,
            pltpu.SemaphoreType.DMA,
        ],
    )
    def agg_kernel(x_hbm, src_hbm, dst_hbm, out_hbm, acc_spmem, rows_vmem,
                   rows_vmem2, gsem_a, gsem_b, ssem_a, ssem_b):
        cid = lax.axis_index("core")
        sid = lax.axis_index("subcore")

        @pl.loop(0, W)
        def _(i):
            for g in range(D // 16):
                rows_vmem[pl.ds(i, 1), pl.ds(g * 16, 16)] = jnp.zeros(
                    (1, 16), jnp.float32)

        zbase = sid * RPT
        for k in range(RPT // W):
            pltpu.sync_copy(rows_vmem, acc_spmem.at[pl.ds(zbase + k * W, W)])
        zrem = RPT % W
        if zrem:
            pltpu.sync_copy(
                rows_vmem.at[pl.ds(0, zrem)],
                acc_spmem.at[pl.ds(zbase + (RPT // W) * W, zrem)])
        plsc.subcore_barrier()

        rows = (rows_vmem, rows_vmem2)
        gsem = (gsem_a, gsem_b)
        ssem = (ssem_a, ssem_b)

        def window(src_idx, dst_idx):
            gd = [None] * KW
            sd = [None] * KW
            gd[0] = pltpu.async_copy(
                x_hbm.at[src_idx.at[0, 0, 0]], rows[0], gsem[0])
            for j in range(KW):
                b = j % 2
                if j + 1 < KW:
                    if j >= 1:
                        sd[j - 1].wait()
                    gd[j + 1] = pltpu.async_copy(
                        x_hbm.at[src_idx.at[0, 0, j + 1]],
                        rows[(j + 1) % 2], gsem[(j + 1) % 2])
                gd[j].wait()
                sd[j] = pltpu.async_copy(
                    rows[b], acc_spmem.at[dst_idx.at[0, 0, j]], ssem[b],
                    add=True)
            sd[KW - 2].wait()
            sd[KW - 1].wait()

        pltpu.emit_pipeline(
            window,
            grid=(NUM_TILES, CPT),
            in_specs=[
                pl.BlockSpec((1, 1, KW, W), lambda t, i: (t, i, 0, 0)),
                pl.BlockSpec((1, 1, KW, W), lambda t, i: (t, i, 0, 0)),
            ],
            core_axis_name=("core", "subcore"),
            dimension_semantics=(pltpu.PARALLEL, pltpu.ARBITRARY),
        )(src_hbm, dst_hbm)

        plsc.subcore_barrier()
        obase = sid * RPT
        pltpu.sync_copy(acc_spmem.at[pl.ds(obase, RPT)],
                        out_hbm.at[cid, pl.ds(obase, RPT)])

    return agg_kernel(x, src4, dst4)


def _mlp_body(x_ref, p_ref, w1t_ref, b1_ref, gamma_ref, beta_ref, w2t_ref,
              b2_ref, o_ref):
    h = x_ref[...] + p_ref[0, pl.ds(0, N)] + p_ref[1, pl.ds(0, N)]
    h1 = jnp.dot(h, w1t_ref[...], preferred_element_type=jnp.float32)
    h1 = h1 + b1_ref[...]
    mean = jnp.mean(h1, axis=0, keepdims=True)
    c = h1 - mean
    var = jnp.mean(c * c, axis=0, keepdims=True)
    hn = c * lax.rsqrt(var + BN_EPS) * gamma_ref[...] + beta_ref[...]
    h2 = jnp.maximum(hn, 0.0)
    o = jnp.dot(h2, w2t_ref[...], preferred_element_type=jnp.float32)
    o = o + b2_ref[...]
    o_ref[...] = jnp.maximum(o, 0.0)


@jax.jit
def _tc_mlp(x, p, w1t, b1, gamma, beta, w2t, b2):
    return pl.pallas_call(
        _mlp_body,
        out_shape=jax.ShapeDtypeStruct((N, D), jnp.float32),
        compiler_params=pltpu.CompilerParams(
            vmem_limit_bytes=100 * 1024 * 1024),
    )(x, p, w1t, b1.reshape(1, D), gamma.reshape(1, D), beta.reshape(1, D),
      w2t, b2.reshape(1, D))


def kernel(x, edge_index, W1, b1, gamma, beta, W2, b2):
    src = edge_index[0].astype(jnp.int32)
    dst = edge_index[1].astype(jnp.int32)
    npad = EPAD - E_IN
    pad_ids = jnp.arange(npad, dtype=jnp.int32)
    src_pad = (pad_ids * 37) % N
    dst_pad = N + pad_ids % (NPAD - N)
    src_full = jnp.concatenate([src, src_pad])
    dst_full = jnp.concatenate([dst, dst_pad])
    p = _sc_aggregate(x, src_full, dst_full)
    return _tc_mlp(x, p, W1.T, b1, gamma, beta, W2.T, b2)
